# baseline (speedup 1.0000x reference)
import functools
import jax
import jax.numpy as jnp
from jax import lax
from jax.experimental import pallas as pl
from jax.experimental.pallas import tpu as pltpu
from jax.experimental.pallas import tpu_sc as plsc

K_SEL = 16
TB = 4


def _colmax(a):
    r = a.shape[0]
    while r > 8:
        a = jnp.maximum(a[: r // 2], a[r // 2 :])
        r //= 2
    return jnp.max(a, axis=0, keepdims=True)


def _colmin(a):
    r = a.shape[0]
    while r > 8:
        a = jnp.minimum(a[: r // 2], a[r // 2 :])
        r //= 2
    return jnp.min(a, axis=0, keepdims=True)


def _topk_body(slots_ref, mask_ref, vals_ref, idx_ref):
    for t in range(slots_ref.shape[0]):
        _one_batch(slots_ref, mask_ref, vals_ref, idx_ref, t)


def _one_batch(slots_ref, mask_ref, vals_ref, idx_ref, t):
    x = slots_ref[t]
    km = mask_ref[t]
    K = x.shape[0]

    sq = jnp.sum(x * x, axis=1, keepdims=True)
    xn = x * (1.0 / jnp.maximum(jnp.sqrt(sq), 1e-12))

    sim = jax.lax.dot_general(
        xn, xn, (((1,), (1,)), ((), ())), preferred_element_type=jnp.float32
    )

    row_i = jax.lax.broadcasted_iota(jnp.int32, (K, K), 0)
    col_j = jax.lax.broadcasted_iota(jnp.int32, (K, K), 1)
    mask2d = km.T * km
    sim = jnp.where(row_i == col_j, 0.0, sim * mask2d)

    bits = jax.lax.bitcast_convert_type(sim, jnp.int32)
    key = jnp.where(bits < 0, bits ^ jnp.int32(0x7FFFFFFF), bits)
    imin = jnp.int32(-2147483648)
    vals_l = []
    idx_l = []
    for _ in range(K_SEL):
        m = _colmax(key)
        eq = key == m
        cand = jnp.where(eq, row_i, K)
        jmin = _colmin(cand)
        key = jnp.where(eq, imin, key)
        vbits = jnp.where(m < 0, m ^ jnp.int32(0x7FFFFFFF), m)
        vals_l.append(jax.lax.bitcast_convert_type(vbits, jnp.float32) * 0.5)
        idx_l.append(jmin)
    vals_ref[t] = jnp.concatenate(vals_l, axis=0)
    idx_ref[t] = jnp.concatenate(idx_l, axis=0)


def _tc_topk(slots, keep_mask):
    B, K, D = slots.shape
    return pl.pallas_call(
        _topk_body,
        grid=(B // TB,),
        in_specs=[
            pl.BlockSpec((TB, K, D), lambda b: (b, 0, 0)),
            pl.BlockSpec((TB, 1, K), lambda b: (b, 0, 0)),
        ],
        out_specs=[
            pl.BlockSpec((TB, K_SEL, K), lambda b: (b, 0, 0)),
            pl.BlockSpec((TB, K_SEL, K), lambda b: (b, 0, 0)),
        ],
        out_shape=[
            jax.ShapeDtypeStruct((B, K_SEL, K), jnp.float32),
            jax.ShapeDtypeStruct((B, K_SEL, K), jnp.int32),
        ],
    )(slots, keep_mask.reshape(B, 1, K))


def _sc_build(vals, idx, B, K):
    mesh = plsc.VectorSubcoreMesh(core_axis_name="c", subcore_axis_name="s")
    info = plsc.get_sparse_core_info()
    nw = info.num_cores * info.num_subcores
    per_w = (B + nw - 1) // nw

    @functools.partial(
        pl.kernel,
        mesh=mesh,
        out_type=jax.ShapeDtypeStruct((B, K * K), jnp.float32),
        scratch_types=[
            pltpu.VMEM((K * K,), jnp.float32),
            pltpu.VMEM((K * K_SEL,), jnp.float32),
            pltpu.VMEM((K * K_SEL,), jnp.int32),
        ],
        compiler_params=pltpu.CompilerParams(use_tc_tiling_on_sc=False, needs_layout_passes=False),
    )
    def scatter_kernel(vals_hbm, idx_hbm, out_hbm, adj_v, vv, iv):
        wid = lax.axis_index("s") * info.num_cores + lax.axis_index("c")
        zero16 = jnp.zeros((16,), jnp.float32)
        ramp = lax.iota(jnp.int32, 16) * K

        for p in range(per_w):
            b = p * nw + wid

            @pl.when(b < B)
            def _do_batch():
                pltpu.sync_copy(vals_hbm.at[b], vv)
                pltpu.sync_copy(idx_hbm.at[b], iv)

                def zero_chunk(i, carry):
                    for c in range(8):
                        adj_v[pl.ds(i * 128 + c * 16, 16)] = zero16
                    return carry

                lax.fori_loop(0, K * K // 128, zero_chunk, 0)

                def scatter_row(i, carry):
                    v = plsc.load_gather(vv, [ramp + i])
                    jv = plsc.load_gather(iv, [ramp + i])
                    plsc.addupdate_scatter(adj_v, [i * K + jv], v)
                    plsc.addupdate_scatter(adj_v, [jv * K + i], v)
                    return carry

                lax.fori_loop(0, K, scatter_row, 0)
                pltpu.sync_copy(adj_v, out_hbm.at[b])

    return scatter_kernel(vals.reshape(B, K * K_SEL), idx.reshape(B, K * K_SEL))


N_CHUNKS = 4


@jax.jit
def kernel(slots, keep_mask):
    B, K, D = slots.shape
    cb = B // N_CHUNKS
    outs = []
    for c in range(N_CHUNKS):
        sl = slots[c * cb:(c + 1) * cb]
        km = keep_mask[c * cb:(c + 1) * cb]
        vals, idx = _tc_topk(sl, km)
        outs.append(_sc_build(vals, idx, cb, K).reshape(cb, K, K))
    return jnp.concatenate(outs, axis=0)


# speedup vs baseline: 1.2801x; 1.2801x over previous
import functools
import jax
import jax.numpy as jnp
from jax import lax
from jax.experimental import pallas as pl
from jax.experimental.pallas import tpu as pltpu
from jax.experimental.pallas import tpu_sc as plsc

K_SEL = 16
TB = 4


def _colmax(a):
    r = a.shape[0]
    while r > 8:
        a = jnp.maximum(a[: r // 2], a[r // 2 :])
        r //= 2
    return jnp.max(a, axis=0, keepdims=True)


def _colmin(a):
    r = a.shape[0]
    while r > 8:
        a = jnp.minimum(a[: r // 2], a[r // 2 :])
        r //= 2
    return jnp.min(a, axis=0, keepdims=True)


def _topk_body(slots_ref, mask_ref, vals_ref, idx_ref):
    for t in range(slots_ref.shape[0]):
        _one_batch(slots_ref, mask_ref, vals_ref, idx_ref, t)


def _one_batch(slots_ref, mask_ref, vals_ref, idx_ref, t):
    x = slots_ref[t]
    km = mask_ref[t]
    K = x.shape[0]

    sq = jnp.sum(x * x, axis=1, keepdims=True)
    xn = x * (1.0 / jnp.maximum(jnp.sqrt(sq), 1e-12))

    sim = jax.lax.dot_general(
        xn, xn, (((1,), (1,)), ((), ())), preferred_element_type=jnp.float32
    )

    row_i = jax.lax.broadcasted_iota(jnp.int32, (K, K), 0)
    col_j = jax.lax.broadcasted_iota(jnp.int32, (K, K), 1)
    mask2d = km.T * km
    sim = jnp.where(row_i == col_j, 0.0, sim * mask2d)

    bits = jax.lax.bitcast_convert_type(sim, jnp.int32)
    key = jnp.where(bits < 0, bits ^ jnp.int32(0x7FFFFFFF), bits)
    imin = jnp.int32(-2147483648)
    iota_w = jax.lax.broadcasted_iota(jnp.int32, (1, K), 1).astype(jnp.float32)
    vals_l = []
    idx_l = []
    for _ in range(K_SEL):
        m = _colmax(key)
        eq = key == m
        key = jnp.where(eq, imin, key)
        eqf = jnp.where(eq, 1.0, 0.0)
        idxf = jax.lax.dot_general(
            iota_w, eqf, (((1,), (0,)), ((), ())),
            preferred_element_type=jnp.float32,
        )
        vbits = jnp.where(m < 0, m ^ jnp.int32(0x7FFFFFFF), m)
        vals_l.append(jax.lax.bitcast_convert_type(vbits, jnp.float32) * 0.5)
        idx_l.append(jnp.minimum(idxf.astype(jnp.int32), K - 1))
    vals_ref[t] = jnp.concatenate(vals_l, axis=0)
    idx_ref[t] = jnp.concatenate(idx_l, axis=0)


def _tc_topk(slots, keep_mask):
    B, K, D = slots.shape
    return pl.pallas_call(
        _topk_body,
        grid=(B // TB,),
        in_specs=[
            pl.BlockSpec((TB, K, D), lambda b: (b, 0, 0)),
            pl.BlockSpec((TB, 1, K), lambda b: (b, 0, 0)),
        ],
        out_specs=[
            pl.BlockSpec((TB, K_SEL, K), lambda b: (b, 0, 0)),
            pl.BlockSpec((TB, K_SEL, K), lambda b: (b, 0, 0)),
        ],
        out_shape=[
            jax.ShapeDtypeStruct((B, K_SEL, K), jnp.float32),
            jax.ShapeDtypeStruct((B, K_SEL, K), jnp.int32),
        ],
    )(slots, keep_mask.reshape(B, 1, K))


def _sc_build(vals, idx, B, K):
    mesh = plsc.VectorSubcoreMesh(core_axis_name="c", subcore_axis_name="s")
    info = plsc.get_sparse_core_info()
    nw = info.num_cores * info.num_subcores
    per_w = (B + nw - 1) // nw

    @functools.partial(
        pl.kernel,
        mesh=mesh,
        out_type=jax.ShapeDtypeStruct((B, K * K), jnp.float32),
        scratch_types=[
            pltpu.VMEM((K * K,), jnp.float32),
            pltpu.VMEM((K * K_SEL,), jnp.float32),
            pltpu.VMEM((K * K_SEL,), jnp.int32),
        ],
        compiler_params=pltpu.CompilerParams(use_tc_tiling_on_sc=False, needs_layout_passes=False),
    )
    def scatter_kernel(vals_hbm, idx_hbm, out_hbm, adj_v, vv, iv):
        wid = lax.axis_index("s") * info.num_cores + lax.axis_index("c")
        zero16 = jnp.zeros((16,), jnp.float32)
        ramp = lax.iota(jnp.int32, 16) * K

        for p in range(per_w):
            b = p * nw + wid

            @pl.when(b < B)
            def _do_batch():
                pltpu.sync_copy(vals_hbm.at[b], vv)
                pltpu.sync_copy(idx_hbm.at[b], iv)

                def zero_chunk(i, carry):
                    for c in range(8):
                        adj_v[pl.ds(i * 128 + c * 16, 16)] = zero16
                    return carry

                lax.fori_loop(0, K * K // 128, zero_chunk, 0)

                def scatter_row(i, carry):
                    v = plsc.load_gather(vv, [ramp + i])
                    jv = plsc.load_gather(iv, [ramp + i])
                    plsc.addupdate_scatter(adj_v, [i * K + jv], v)
                    plsc.addupdate_scatter(adj_v, [jv * K + i], v)
                    return carry

                lax.fori_loop(0, K, scatter_row, 0)
                pltpu.sync_copy(adj_v, out_hbm.at[b])

    return scatter_kernel(vals.reshape(B, K * K_SEL), idx.reshape(B, K * K_SEL))


N_CHUNKS = 1


@jax.jit
def kernel(slots, keep_mask):
    B, K, D = slots.shape
    cb = B // N_CHUNKS
    outs = []
    for c in range(N_CHUNKS):
        sl = slots[c * cb:(c + 1) * cb]
        km = keep_mask[c * cb:(c + 1) * cb]
        vals, idx = _tc_topk(sl, km)
        outs.append(_sc_build(vals, idx, cb, K).reshape(cb, K, K))
    return jnp.concatenate(outs, axis=0)


# speedup vs baseline: 1.3907x; 1.0864x over previous
import functools
import jax
import jax.numpy as jnp
from jax import lax
from jax.experimental import pallas as pl
from jax.experimental.pallas import tpu as pltpu
from jax.experimental.pallas import tpu_sc as plsc

K_SEL = 16
TB = 8


def _colmax(a):
    r = a.shape[0]
    while r > 8:
        a = jnp.maximum(a[: r // 2], a[r // 2 :])
        r //= 2
    return jnp.max(a, axis=0, keepdims=True)


def _topk_body(slots_ref, mask_ref, vals_ref, idx_ref):
    for t in range(slots_ref.shape[0]):
        _one_batch(slots_ref, mask_ref, vals_ref, idx_ref, t)


def _one_batch(slots_ref, mask_ref, vals_ref, idx_ref, t):
    x = slots_ref[t]
    km = mask_ref[t]
    K = x.shape[0]

    sq = jnp.sum(x * x, axis=1, keepdims=True)
    xn = x * (1.0 / jnp.maximum(jnp.sqrt(sq), 1e-12))

    sim = jax.lax.dot_general(
        xn, xn, (((1,), (1,)), ((), ())), preferred_element_type=jnp.float32
    )

    row_i = jax.lax.broadcasted_iota(jnp.int32, (K, K), 0)
    col_j = jax.lax.broadcasted_iota(jnp.int32, (K, K), 1)
    mask2d = km.T * km
    sim = jnp.where(row_i == col_j, 0.0, sim * mask2d)

    bits = jax.lax.bitcast_convert_type(sim, jnp.int32)
    key = jnp.where(bits < 0, bits ^ jnp.int32(0x7FFFFFFF), bits)
    imin = jnp.int32(-2147483648)
    iota_w = jax.lax.broadcasted_iota(jnp.int32, (1, K), 1).astype(jnp.float32)
    vals_l = []
    idx_l = []
    for _ in range(K_SEL):
        m = _colmax(key)
        eq = key == m
        key = jnp.where(eq, imin, key)
        eqf = jnp.where(eq, 1.0, 0.0)
        idxf = jax.lax.dot_general(
            iota_w, eqf, (((1,), (0,)), ((), ())),
            preferred_element_type=jnp.float32,
        )
        vbits = jnp.where(m < 0, m ^ jnp.int32(0x7FFFFFFF), m)
        vals_l.append(jax.lax.bitcast_convert_type(vbits, jnp.float32) * 0.5)
        idx_l.append(jnp.minimum(idxf.astype(jnp.int32), K - 1))
    vals_ref[t] = jnp.concatenate(vals_l, axis=0)
    idx_ref[t] = jnp.concatenate(idx_l, axis=0)


def _tc_topk(slots, keep_mask):
    B, K, D = slots.shape
    return pl.pallas_call(
        _topk_body,
        grid=(B // TB,),
        in_specs=[
            pl.BlockSpec((TB, K, D), lambda b: (b, 0, 0)),
            pl.BlockSpec((TB, 1, K), lambda b: (b, 0, 0)),
        ],
        out_specs=[
            pl.BlockSpec((TB, K_SEL, K), lambda b: (b, 0, 0)),
            pl.BlockSpec((TB, K_SEL, K), lambda b: (b, 0, 0)),
        ],
        out_shape=[
            jax.ShapeDtypeStruct((B, K_SEL, K), jnp.float32),
            jax.ShapeDtypeStruct((B, K_SEL, K), jnp.int32),
        ],
    )(slots, keep_mask.reshape(B, 1, K))


def _sc_build(vals, idx, B, K):
    mesh = plsc.VectorSubcoreMesh(core_axis_name="c", subcore_axis_name="s")
    info = plsc.get_sparse_core_info()
    nw = info.num_cores * info.num_subcores
    per_w = (B + nw - 1) // nw

    @functools.partial(
        pl.kernel,
        mesh=mesh,
        out_type=jax.ShapeDtypeStruct((B, K * K), jnp.float32),
        scratch_types=[
            pltpu.VMEM((K * K,), jnp.float32),
            pltpu.VMEM((K * K_SEL,), jnp.float32),
            pltpu.VMEM((K * K_SEL,), jnp.int32),
        ],
        compiler_params=pltpu.CompilerParams(use_tc_tiling_on_sc=False, needs_layout_passes=False),
    )
    def scatter_kernel(vals_hbm, idx_hbm, out_hbm, adj_v, vv, iv):
        wid = lax.axis_index("s") * info.num_cores + lax.axis_index("c")
        zero16 = jnp.zeros((16,), jnp.float32)
        lane = lax.iota(jnp.int32, 16)
        ramp_k = lane * K

        for p in range(per_w):
            b = p * nw + wid

            @pl.when(b < B)
            def _do_batch():
                pltpu.sync_copy(vals_hbm.at[b], vv)
                pltpu.sync_copy(idx_hbm.at[b], iv)

                def zero_row(i, carry):
                    for c in range(K // 16):
                        adj_v[pl.ds(i * K + c * 16, 16)] = zero16
                    return carry

                lax.fori_loop(0, K, zero_row, 0)

                def scatter_round(r, carry):
                    base = r * K
                    for ib in range(K // 16):
                        v = vv[pl.ds(base + ib * 16, 16)]
                        jv = iv[pl.ds(base + ib * 16, 16)]
                        plsc.addupdate_scatter(
                            adj_v, [ramp_k + (ib * 16 * K) + jv], v)
                        plsc.addupdate_scatter(
                            adj_v, [jv * K + (lane + ib * 16)], v)
                    return carry

                lax.fori_loop(0, K_SEL, scatter_round, 0)
                pltpu.sync_copy(adj_v, out_hbm.at[b])

    return scatter_kernel(vals.reshape(B, K * K_SEL), idx.reshape(B, K * K_SEL))


@jax.jit
def kernel(slots, keep_mask):
    B, K, D = slots.shape
    vals, idx = _tc_topk(slots, keep_mask)
    return _sc_build(vals, idx, B, K).reshape(B, K, K)
